# Initial kernel scaffold; baseline (speedup 1.0000x reference)
import jax
import jax.numpy as jnp
from jax.experimental import pallas as pl


def kernel(x, edge_index, W1, b1, W2, b2):
    raise NotImplementedError("write your pallas kernel here")


# speedup vs baseline: 12.6037x; 12.6037x over previous
import functools

import jax
import jax.numpy as jnp
from jax import lax
from jax.experimental import pallas as pl
from jax.experimental.pallas import tpu as pltpu
from jax.experimental.pallas import tpu_sc as plsc

_N = 10000
_E = 320000
_D = 128

_NC = 2
_NS = 16
_NW = _NC * _NS
_EPT = _E // _NW
_CH = 80
_NCHUNK = _EPT // _CH
_NPAD = 10240
_RPT = _NPAD // _NS
_ZR = 128

_mesh = plsc.VectorSubcoreMesh(core_axis_name="c", subcore_axis_name="s")


def _zero_fill(ref, rows, width):
    z16 = jnp.zeros((16,), jnp.float32)

    def body(i, _):
        for j in range(width // 16):
            ref[i, pl.ds(16 * j, 16)] = z16
        return 0

    lax.fori_loop(0, rows, body, 0)


def _count_body(dst_hbm, cnt_hbm, ones_v, idx_v, zbuf, cnt_sh):
    c = lax.axis_index("c")
    s = lax.axis_index("s")
    g = c * _NS + s

    one16 = jnp.ones((16,), jnp.float32)

    def fill(i, _):
        for j in range(_D // 16):
            ones_v[i, pl.ds(16 * j, 16)] = one16
        return 0

    lax.fori_loop(0, _CH, fill, 0)
    _zero_fill(zbuf, _ZR, _D)

    for j in range(_RPT // _ZR):
        pltpu.sync_copy(zbuf, cnt_sh.at[pl.ds(s * _RPT + j * _ZR, _ZR)])
    plsc.subcore_barrier()

    base = g * _EPT

    def chunk(i, _):
        pltpu.sync_copy(dst_hbm.at[pl.ds(base + i * _CH, _CH)], idx_v)
        pltpu.sync_copy(ones_v, cnt_sh.at[idx_v], add=True)
        return 0

    lax.fori_loop(0, _NCHUNK, chunk, 0)
    plsc.subcore_barrier()

    for j in range(_RPT // _ZR):
        r0 = s * _RPT + j * _ZR
        pltpu.sync_copy(cnt_sh.at[pl.ds(r0, _ZR)], cnt_hbm.at[c, pl.ds(r0, _ZR)])


_sc_count = functools.partial(
    pl.kernel,
    out_type=jax.ShapeDtypeStruct((_NC, _NPAD, _D), jnp.float32),
    mesh=_mesh,
    scratch_types=[
        pltpu.VMEM((_CH, _D), jnp.float32),
        pltpu.VMEM((_CH,), jnp.int32),
        pltpu.VMEM((_ZR, _D), jnp.float32),
        pltpu.VMEM_SHARED((_NPAD, _D), jnp.float32),
    ],
)(_count_body)


def _prop_body(hs_hbm, src_hbm, dst_hbm, out_hbm,
               idx_s, idx_d, rows, zbuf, acc_sh, sem):
    c = lax.axis_index("c")
    s = lax.axis_index("s")
    g = c * _NS + s

    _zero_fill(zbuf, _ZR, _D)
    for j in range(_RPT // _ZR):
        pltpu.sync_copy(zbuf, acc_sh.at[pl.ds(s * _RPT + j * _ZR, _ZR)])
    plsc.subcore_barrier()

    base = g * _EPT

    def chunk(i, _):
        off = base + i * _CH
        pltpu.sync_copy(src_hbm.at[pl.ds(off, _CH)], idx_s)
        pltpu.sync_copy(dst_hbm.at[pl.ds(off, _CH)], idx_d)
        pltpu.async_copy(hs_hbm.at[idx_s], rows, sem).wait()
        pltpu.sync_copy(rows, acc_sh.at[idx_d], add=True)
        return 0

    lax.fori_loop(0, _NCHUNK, chunk, 0)
    plsc.subcore_barrier()

    for j in range(_RPT // _ZR):
        r0 = s * _RPT + j * _ZR
        pltpu.sync_copy(acc_sh.at[pl.ds(r0, _ZR)], out_hbm.at[c, pl.ds(r0, _ZR)])


_sc_prop = functools.partial(
    pl.kernel,
    out_type=jax.ShapeDtypeStruct((_NC, _NPAD, _D), jnp.float32),
    mesh=_mesh,
    scratch_types=[
        pltpu.VMEM((_CH,), jnp.int32),
        pltpu.VMEM((_CH,), jnp.int32),
        pltpu.VMEM((_CH, _D), jnp.float32),
        pltpu.VMEM((_ZR, _D), jnp.float32),
        pltpu.VMEM_SHARED((_NPAD, _D), jnp.float32),
        pltpu.SemaphoreType.DMA,
    ],
)(_prop_body)


_BLK = 2000
_GRID = _N // _BLK


def _tc1_body(cnt_ref, x_ref, w1_ref, hs_ref, dis_ref):
    cnt = cnt_ref[0] + cnt_ref[1]
    deg = jnp.max(cnt, axis=1, keepdims=True) + 1.0
    dis = lax.rsqrt(jnp.maximum(deg, 1.0))
    hs_ref[...] = jnp.dot(x_ref[...], w1_ref[...],
                          preferred_element_type=jnp.float32) * dis
    dis_ref[...] = dis


def _tc1(cnt, x, W1):
    return pl.pallas_call(
        _tc1_body,
        grid=(_GRID,),
        in_specs=[
            pl.BlockSpec((_NC, _BLK, _D), lambda i: (0, i, 0)),
            pl.BlockSpec((_BLK, _D), lambda i: (i, 0)),
            pl.BlockSpec((_D, _D), lambda i: (0, 0)),
        ],
        out_specs=[
            pl.BlockSpec((_BLK, _D), lambda i: (i, 0)),
            pl.BlockSpec((_BLK, 1), lambda i: (i, 0)),
        ],
        out_shape=[
            jax.ShapeDtypeStruct((_N, _D), jnp.float32),
            jax.ShapeDtypeStruct((_N, 1), jnp.float32),
        ],
    )(cnt, x, W1)


def _tc2_body(acc_ref, hs1_ref, dis_ref, b1_ref, w2_ref, hs2_ref):
    dis = dis_ref[...]
    h1 = (acc_ref[0] + acc_ref[1] + hs1_ref[...]) * dis + b1_ref[...]
    h1 = jnp.maximum(h1, 0.0)
    hs2_ref[...] = jnp.dot(h1, w2_ref[...],
                           preferred_element_type=jnp.float32) * dis


def _tc2(acc, hs1, dis, b1, W2):
    return pl.pallas_call(
        _tc2_body,
        grid=(_GRID,),
        in_specs=[
            pl.BlockSpec((_NC, _BLK, _D), lambda i: (0, i, 0)),
            pl.BlockSpec((_BLK, _D), lambda i: (i, 0)),
            pl.BlockSpec((_BLK, 1), lambda i: (i, 0)),
            pl.BlockSpec((1, _D), lambda i: (0, 0)),
            pl.BlockSpec((_D, _D), lambda i: (0, 0)),
        ],
        out_specs=pl.BlockSpec((_BLK, _D), lambda i: (i, 0)),
        out_shape=jax.ShapeDtypeStruct((_N, _D), jnp.float32),
    )(acc, hs1, dis, b1, W2)


def _tc3_body(acc_ref, hs2_ref, dis_ref, b2_ref, z_ref):
    z_ref[...] = ((acc_ref[0] + acc_ref[1] + hs2_ref[...]) * dis_ref[...]
                  + b2_ref[...])


def _tc3(acc, hs2, dis, b2):
    return pl.pallas_call(
        _tc3_body,
        grid=(_GRID,),
        in_specs=[
            pl.BlockSpec((_NC, _BLK, _D), lambda i: (0, i, 0)),
            pl.BlockSpec((_BLK, _D), lambda i: (i, 0)),
            pl.BlockSpec((_BLK, 1), lambda i: (i, 0)),
            pl.BlockSpec((1, _D), lambda i: (0, 0)),
        ],
        out_specs=pl.BlockSpec((_BLK, _D), lambda i: (i, 0)),
        out_shape=jax.ShapeDtypeStruct((_N, _D), jnp.float32),
    )(acc, hs2, dis, b2)


def kernel(x, edge_index, W1, b1, W2, b2):
    src = edge_index[0].astype(jnp.int32)
    dst = edge_index[1].astype(jnp.int32)
    cnt = _sc_count(dst)
    hs1, dis = _tc1(cnt, x, W1)
    acc1 = _sc_prop(hs1, src, dst)
    hs2 = _tc2(acc1, hs1, dis, b1.reshape(1, _D), W2)
    acc2 = _sc_prop(hs2, src, dst)
    return _tc3(acc2, hs2, dis, b2.reshape(1, _D))
